# Initial kernel scaffold; baseline (speedup 1.0000x reference)
import jax
import jax.numpy as jnp
from jax.experimental import pallas as pl


def kernel(x, edge_index, Wl0, bl0, Wr0, Wl1, bl1, Wr1, Wl2, bl2, Wr2):
    raise NotImplementedError("write your pallas kernel here")


# speedup vs baseline: 3.2455x; 3.2455x over previous
import functools

import jax
import jax.numpy as jnp
from jax import lax
from jax.experimental import pallas as pl
from jax.experimental.pallas import tpu as pltpu
from jax.experimental.pallas import tpu_sc as plsc

N = 10000
NP = 10240
E = 160000
D = 256
DH = D // 2
NS = 16
EPT = E // NS
CH = 80
NCHUNK = EPT // CH
RPT = NP // NS
RCH = 128
NRCH = RPT // RCH
BN = 1024


def _agg_body(x_hbm, src_hbm, dst_hbm, out_hbm, cnt_hbm,
              acc_sh, cacc_sh, sidx_v, didx_v, rows_v, zw_v, zc_v, ones_v,
              with_count):
    c = lax.axis_index("c")
    s = lax.axis_index("s")
    row0 = s * RPT

    def zrow(r, _):
        for j in range(DH // 16):
            zw_v[r, pl.ds(j * 16, 16)] = jnp.zeros((16,), jnp.float32)
        return 0
    lax.fori_loop(0, RCH, zrow, 0)
    for j in range(NRCH):
        pltpu.sync_copy(zw_v, acc_sh.at[pl.ds(row0 + j * RCH, RCH)])
    if with_count:
        def zcrow(r, _):
            zc_v[r, :] = jnp.zeros((16,), jnp.float32)
            return 0
        lax.fori_loop(0, RPT, zcrow, 0)
        pltpu.sync_copy(zc_v, cacc_sh.at[pl.ds(row0, RPT)])

        def onesrow(r, _):
            ones_v[r, :] = jnp.ones((16,), jnp.float32)
            return 0
        lax.fori_loop(0, CH, onesrow, 0)
    plsc.subcore_barrier()

    ebase = s * EPT

    def chunk(k, _):
        e0 = ebase + k * CH
        pltpu.sync_copy(src_hbm.at[pl.ds(e0, CH)], sidx_v)
        pltpu.sync_copy(dst_hbm.at[pl.ds(e0, CH)], didx_v)
        pltpu.sync_copy(x_hbm.at[c].at[sidx_v], rows_v)
        pltpu.sync_copy(rows_v, acc_sh.at[didx_v], add=True)
        if with_count:
            pltpu.sync_copy(ones_v, cacc_sh.at[didx_v], add=True)
        return 0
    lax.fori_loop(0, NCHUNK, chunk, 0)
    plsc.subcore_barrier()

    for j in range(NRCH):
        r0 = row0 + j * RCH
        pltpu.sync_copy(acc_sh.at[pl.ds(r0, RCH)], zw_v)
        pltpu.sync_copy(zw_v, out_hbm.at[c].at[pl.ds(r0, RCH)])
    if with_count:
        pltpu.sync_copy(cacc_sh.at[pl.ds(row0, RPT)], zc_v)
        pltpu.sync_copy(zc_v, cnt_hbm.at[c].at[pl.ds(row0, RPT)])


def _make_agg(with_count):
    mesh = plsc.VectorSubcoreMesh(core_axis_name="c", subcore_axis_name="s")
    out_type = [jax.ShapeDtypeStruct((2, NP, DH), jnp.float32)]
    if with_count:
        out_type.append(jax.ShapeDtypeStruct((2, NP, 16), jnp.float32))
    scratch = [
        pltpu.VMEM_SHARED((NP, DH), jnp.float32),
        pltpu.VMEM_SHARED((NP, 16), jnp.float32),
        pltpu.VMEM((CH,), jnp.int32),
        pltpu.VMEM((CH,), jnp.int32),
        pltpu.VMEM((CH, DH), jnp.float32),
        pltpu.VMEM((RCH, DH), jnp.float32),
        pltpu.VMEM((RPT, 16), jnp.float32),
        pltpu.VMEM((CH, 16), jnp.float32),
    ]

    @functools.partial(
        pl.kernel, mesh=mesh, out_type=out_type, scratch_types=scratch,
        compiler_params=pltpu.CompilerParams(use_tc_tiling_on_sc=False))
    def k(x_hbm, src_hbm, dst_hbm, *rest):
        if with_count:
            out_hbm, cnt_hbm = rest[0], rest[1]
            scr = rest[2:]
        else:
            out_hbm, cnt_hbm = rest[0], None
            scr = rest[1:]
        _agg_body(x_hbm, src_hbm, dst_hbm, out_hbm, cnt_hbm, *scr,
                  with_count=with_count)

    return k


_agg_with_count = _make_agg(True)
_agg = _make_agg(False)


def _scale_body(s_ref, cnt_ref, out_ref):
    invc = 1.0 / jnp.maximum(cnt_ref[0, :, 0:1], 1.0)
    out_ref[0] = s_ref[0] * invc
    out_ref[1] = s_ref[1] * invc


def _scale(s, cnt):
    grid = (NP // BN,)
    return pl.pallas_call(
        _scale_body,
        grid=grid,
        in_specs=[
            pl.BlockSpec((2, BN, DH), lambda i: (0, i, 0)),
            pl.BlockSpec((1, BN, 16), lambda i: (0, i, 0)),
        ],
        out_specs=pl.BlockSpec((2, BN, DH), lambda i: (0, i, 0)),
        out_shape=jax.ShapeDtypeStruct((2, NP, DH), jnp.float32),
    )(s, cnt)


def _mm_body(s2_ref, cnt_ref, h_ref, wl_ref, bl_ref, wr_ref, out_ref,
             *, act, split_out):
    invc = 1.0 / jnp.maximum(cnt_ref[0, :, 0:1], 1.0)
    m2 = jnp.concatenate([s2_ref[0], s2_ref[1]], axis=1) * invc
    h = jnp.concatenate([h_ref[0], h_ref[1]], axis=1)
    dn = (((1,), (1,)), ((), ()))
    res = lax.dot_general(m2, wl_ref[...], dn,
                          preferred_element_type=jnp.float32)
    res = res + bl_ref[...]
    res = res + lax.dot_general(h, wr_ref[...], dn,
                                preferred_element_type=jnp.float32)
    if act:
        res = jnp.maximum(res, 0.0)
    if split_out:
        out_ref[0] = res[:, :DH]
        out_ref[1] = res[:, DH:]
    else:
        out_ref[...] = res


def _mm(s2, cnt, h, wl, bl, wr, act, split_out):
    grid = (NP // BN,)
    if split_out:
        out_spec = pl.BlockSpec((2, BN, DH), lambda i: (0, i, 0))
        out_shape = jax.ShapeDtypeStruct((2, NP, DH), jnp.float32)
    else:
        out_spec = pl.BlockSpec((BN, D), lambda i: (i, 0))
        out_shape = jax.ShapeDtypeStruct((NP, D), jnp.float32)
    return pl.pallas_call(
        functools.partial(_mm_body, act=act, split_out=split_out),
        grid=grid,
        in_specs=[
            pl.BlockSpec((2, BN, DH), lambda i: (0, i, 0)),
            pl.BlockSpec((1, BN, 16), lambda i: (0, i, 0)),
            pl.BlockSpec((2, BN, DH), lambda i: (0, i, 0)),
            pl.BlockSpec((D, D), lambda i: (0, 0)),
            pl.BlockSpec((1, D), lambda i: (0, 0)),
            pl.BlockSpec((D, D), lambda i: (0, 0)),
        ],
        out_specs=out_spec,
        out_shape=out_shape,
    )(s2, cnt, h, wl, bl, wr)


def kernel(x, edge_index, Wl0, bl0, Wr0, Wl1, bl1, Wr1, Wl2, bl2, Wr2):
    src = edge_index[0].astype(jnp.int32)
    dst = edge_index[1].astype(jnp.int32)
    h = jnp.stack([x[:, :DH], x[:, DH:]])
    h = jnp.pad(h, ((0, 0), (0, NP - N), (0, 0)))
    weights = [(Wl0, bl0, Wr0), (Wl1, bl1, Wr1), (Wl2, bl2, Wr2)]

    s1, cnt = _agg_with_count(h, src, dst)
    for i, (wl, bl, wr) in enumerate(weights):
        if i > 0:
            (s1,) = _agg(h, src, dst)
        m1 = _scale(s1, cnt)
        (s2,) = _agg(m1, src, dst)
        last = i == len(weights) - 1
        h = _mm(s2, cnt, h, wl, bl.reshape(1, D), wr,
                act=not last, split_out=not last)
    return h[:N]
